# baseline (speedup 1.0000x reference)
import functools

import jax
import jax.numpy as jnp
from jax import lax
from jax.experimental import pallas as pl
from jax.experimental.pallas import tpu as pltpu
from jax.experimental.pallas import tpu_sc as plsc

B = 1024
K = 128
E = 256
V = 100000
TOPK = 25
VT = 2048
NT = 49
VP = NT * VT
G = 128
NG = VP // G
GPT = VT // G
M = K * TOPK


def _k1_body(te_ref, we_ref, e_ref, gmax_ref, z_ref):
    i = pl.program_id(0)
    wt = jnp.dot(te_ref[...], we_ref[...], preferred_element_type=jnp.float32)
    col = i * VT + lax.broadcasted_iota(jnp.int32, (K, VT), 1)
    e = jnp.where(col < V, jnp.exp(wt), 0.0)
    gi = lax.broadcasted_iota(jnp.int32, (K, GPT), 1)
    gm = jnp.zeros((K, GPT), jnp.float32)
    for g in range(GPT):
        seg = e[:, g * G:(g + 1) * G]
        e_ref[:, g, :] = seg
        m = jnp.max(seg, axis=1, keepdims=True)
        gm = jnp.where(gi == g, m, gm)
    gmax_ref[...] = gm[None]
    zp = jnp.sum(e, axis=1, keepdims=True)

    @pl.when(i == 0)
    def _():
        z_ref[...] = zp

    @pl.when(i > 0)
    def _():
        z_ref[...] = z_ref[...] + zp


def _k2_body(gm_ref, gid_ref):
    g = gm_ref[...]
    gi = lax.broadcasted_iota(jnp.int32, (K, NG), 1)
    oi = lax.broadcasted_iota(jnp.int32, (K, TOPK), 1)
    gid = jnp.zeros((K, TOPK), jnp.int32)
    for j in range(TOPK):
        m = jnp.max(g, axis=1, keepdims=True)
        pos = jnp.min(jnp.where(g == m, gi, 2**30), axis=1, keepdims=True)
        gid = jnp.where(oi == j, pos, gid)
        g = jnp.where(gi == pos, -1.0, g)
    gid_ref[...] = gid


def _k4a_body(cand_ref, cols_ref, s_ref, p_ref):
    c = cand_ref[...]
    colv = cols_ref[...]
    oi = lax.broadcasted_iota(jnp.int32, (K, TOPK), 1)
    vals = jnp.zeros((K, TOPK), jnp.float32)
    poss = jnp.zeros((K, TOPK), jnp.int32)
    for j in range(TOPK):
        m = jnp.max(c, axis=1, keepdims=True)
        pos = jnp.min(jnp.where(c == m, colv, 2**30), axis=1, keepdims=True)
        vals = jnp.where(oi == j, m, vals)
        poss = jnp.where(oi == j, pos, poss)
        c = jnp.where(colv == pos, -1.0, c)
    s_ref[...] = vals / jnp.sum(vals, axis=1, keepdims=True)
    p_ref[...] = poss


def _k4b_body(s_ref, p_ref, sf_ref, pf_ref, out_ref):
    S = s_ref[...]
    Pp = p_ref[...]
    sf = sf_ref[...]
    pf = pf_ref[...]
    acc = jnp.zeros((K, M), jnp.float32)
    for a in range(TOPK):
        pa = Pp[:, a:a + 1]
        sa = S[:, a:a + 1]
        acc = acc + jnp.where(pf == pa, sa * sf, 0.0)
    r = (lax.broadcasted_iota(jnp.int32, (M, K), 0) // TOPK ==
         lax.broadcasted_iota(jnp.int32, (M, K), 1)).astype(jnp.float32)
    gram = jnp.dot(acc, r, preferred_element_type=jnp.float32)
    ncol = jnp.sum(S * S, axis=1, keepdims=True)
    nrow = jnp.dot(sf * sf, r, preferred_element_type=jnp.float32)
    d = jnp.maximum(ncol + nrow - 2.0 * gram, 0.0)
    t = jnp.sum(d, axis=1, keepdims=True)
    out_ref[...] = 0.5 * jnp.sum(t, axis=0, keepdims=True) / (K * K)


def _k5_body(alpha_ref, z_ref, e_ref, db0_ref, db1_ref, re_ref, th_ref):
    i = pl.program_id(0)
    a = alpha_ref[...]
    mx = jnp.max(a, axis=1, keepdims=True)
    ex = jnp.exp(a - mx)
    th = ex / jnp.sum(ex, axis=1, keepdims=True)

    @pl.when(i == 0)
    def _():
        th_ref[...] = th

    tp = th * (1.0 / z_ref[...])
    ev = jnp.concatenate([e_ref[:, g, :] for g in range(GPT)], axis=1)
    H = B // 2
    col = i * VT + lax.broadcasted_iota(jnp.int32, (H, VT), 1)
    valid = col < V
    accs = []
    for h, db_ref in ((0, db0_ref), (1, db1_ref)):
        p = jnp.dot(tp[h * H:(h + 1) * H, :].astype(jnp.bfloat16),
                    ev.astype(jnp.bfloat16),
                    preferred_element_type=jnp.float32)
        lg = jnp.log(jnp.where(valid, p, 1.0))
        contrib = lg * jnp.where(valid, db_ref[...], 0.0)
        accs.append(jnp.sum(contrib, axis=1, keepdims=True))
    acc = jnp.concatenate(accs, axis=0)

    @pl.when(i == 0)
    def _():
        re_ref[...] = -acc

    @pl.when(i > 0)
    def _():
        re_ref[...] = re_ref[...] - acc


def _sc_gather(table, idx, nrows, nc):
    nw = nc * 16
    bpw = nrows // nw
    mesh = plsc.VectorSubcoreMesh(core_axis_name="c", subcore_axis_name="s")

    @functools.partial(
        pl.kernel, mesh=mesh,
        out_type=jax.ShapeDtypeStruct((nrows, G), jnp.float32),
        scratch_types=[
            pltpu.VMEM((bpw,), jnp.int32),
            pltpu.VMEM((bpw, G), jnp.float32),
            pltpu.SemaphoreType.DMA,
        ],
    )
    def k(table_hbm, idx_hbm, out_hbm, idx_v, rows_v, sem):
        wid = lax.axis_index("s") * nc + lax.axis_index("c")
        base = wid * bpw
        pltpu.sync_copy(idx_hbm.at[pl.ds(base, bpw)], idx_v)
        pltpu.async_copy(table_hbm.at[idx_v], rows_v, sem).wait()
        pltpu.sync_copy(rows_v, out_hbm.at[pl.ds(base, bpw)])

    return k(table, idx)


def kernel(alpha, doc_bow, topic_embeddings_mat, word_embeddings_mat):
    te = topic_embeddings_mat
    we = word_embeddings_mat

    e, gmax3, z = pl.pallas_call(
        _k1_body,
        grid=(NT,),
        in_specs=[
            pl.BlockSpec((K, E), lambda i: (0, 0)),
            pl.BlockSpec((E, VT), lambda i: (0, i)),
        ],
        out_specs=[
            pl.BlockSpec((K, GPT, G), lambda i: (0, i, 0)),
            pl.BlockSpec((1, K, GPT), lambda i: (i, 0, 0)),
            pl.BlockSpec((K, 1), lambda i: (0, 0)),
        ],
        out_shape=[
            jax.ShapeDtypeStruct((K, NG, G), jnp.float32),
            jax.ShapeDtypeStruct((NT, K, GPT), jnp.float32),
            jax.ShapeDtypeStruct((K, 1), jnp.float32),
        ],
    )(te, we)

    gmax = gmax3.transpose(1, 0, 2).reshape(K, NG)
    gid = pl.pallas_call(
        _k2_body,
        out_shape=jax.ShapeDtypeStruct((K, TOPK), jnp.int32),
    )(gmax)

    info = plsc.get_sparse_core_info()
    nc = info.num_cores
    align = 8 * nc * 16
    nrows = ((M + align - 1) // align) * align
    rowid = jnp.arange(K, dtype=jnp.int32)[:, None] * NG + gid
    idx = jnp.concatenate(
        [rowid.reshape(-1), jnp.zeros((nrows - M,), jnp.int32)])
    table = e.reshape(K * NG, G)
    cand = _sc_gather(table, idx, nrows, nc)[:M].reshape(K, TOPK * G)
    cols = (gid[:, :, None] * G +
            jnp.arange(G, dtype=jnp.int32)[None, None, :]).reshape(K, TOPK * G)

    S, poss = pl.pallas_call(
        _k4a_body,
        out_shape=[
            jax.ShapeDtypeStruct((K, TOPK), jnp.float32),
            jax.ShapeDtypeStruct((K, TOPK), jnp.int32),
        ],
    )(cand, cols)

    stdr = pl.pallas_call(
        _k4b_body,
        out_shape=jax.ShapeDtypeStruct((1, 1), jnp.float32),
    )(S, poss, S.reshape(1, M), poss.reshape(1, M))

    zrow = z.reshape(1, K)
    re, theta = pl.pallas_call(
        _k5_body,
        grid=(NT,),
        in_specs=[
            pl.BlockSpec((B, K), lambda i: (0, 0)),
            pl.BlockSpec((1, K), lambda i: (0, 0)),
            pl.BlockSpec((K, GPT, G), lambda i: (0, i, 0)),
            pl.BlockSpec((B // 2, VT), lambda i: (0, i)),
            pl.BlockSpec((B // 2, VT), lambda i: (1, i)),
        ],
        out_specs=[
            pl.BlockSpec((B, 1), lambda i: (0, 0)),
            pl.BlockSpec((B, K), lambda i: (0, 0)),
        ],
        out_shape=[
            jax.ShapeDtypeStruct((B, 1), jnp.float32),
            jax.ShapeDtypeStruct((B, K), jnp.float32),
        ],
    )(alpha, zrow, e, doc_bow, doc_bow)

    return (re.reshape(B), stdr.reshape(()), theta)


# speedup vs baseline: 1.0127x; 1.0127x over previous
import functools

import jax
import jax.numpy as jnp
from jax import lax
from jax.experimental import pallas as pl
from jax.experimental.pallas import tpu as pltpu
from jax.experimental.pallas import tpu_sc as plsc

B = 1024
K = 128
E = 256
V = 100000
TOPK = 25
VT = 2048
NT = 49
VP = NT * VT
G = 128
NG = VP // G
GPT = VT // G
M = K * TOPK
VT5 = 4096
NT5 = 25
GPT5 = VT5 // G


def _k1_body(te_ref, we_ref, e_ref, gmax_ref, z_ref):
    i = pl.program_id(0)
    wt = jnp.dot(te_ref[...], we_ref[...], preferred_element_type=jnp.float32)
    col = i * VT + lax.broadcasted_iota(jnp.int32, (K, VT), 1)
    e = jnp.where(col < V, jnp.exp(wt), 0.0)
    gi = lax.broadcasted_iota(jnp.int32, (K, GPT), 1)
    gm = jnp.zeros((K, GPT), jnp.float32)
    for g in range(GPT):
        seg = e[:, g * G:(g + 1) * G]
        e_ref[:, g, :] = seg
        m = jnp.max(seg, axis=1, keepdims=True)
        gm = jnp.where(gi == g, m, gm)
    gmax_ref[...] = gm[None]
    zp = jnp.sum(e, axis=1, keepdims=True)

    @pl.when(i == 0)
    def _():
        z_ref[...] = zp

    @pl.when(i > 0)
    def _():
        z_ref[...] = z_ref[...] + zp


def _k2_body(gm_ref, gid_ref):
    g = gm_ref[...]
    gi = lax.broadcasted_iota(jnp.int32, (K, NG), 1)
    oi = lax.broadcasted_iota(jnp.int32, (K, TOPK), 1)
    gid = jnp.zeros((K, TOPK), jnp.int32)
    for j in range(TOPK):
        m = jnp.max(g, axis=1, keepdims=True)
        pos = jnp.min(jnp.where(g == m, gi, 2**30), axis=1, keepdims=True)
        gid = jnp.where(oi == j, pos, gid)
        g = jnp.where(gi == pos, -1.0, g)
    gid_ref[...] = gid


def _k4a_body(cand_ref, cols_ref, s_ref, p_ref):
    c = cand_ref[...]
    colv = cols_ref[...]
    oi = lax.broadcasted_iota(jnp.int32, (K, TOPK), 1)
    vals = jnp.zeros((K, TOPK), jnp.float32)
    poss = jnp.zeros((K, TOPK), jnp.int32)
    for j in range(TOPK):
        m = jnp.max(c, axis=1, keepdims=True)
        pos = jnp.min(jnp.where(c == m, colv, 2**30), axis=1, keepdims=True)
        vals = jnp.where(oi == j, m, vals)
        poss = jnp.where(oi == j, pos, poss)
        c = jnp.where(colv == pos, -1.0, c)
    s_ref[...] = vals / jnp.sum(vals, axis=1, keepdims=True)
    p_ref[...] = poss


def _k4b_body(s_ref, p_ref, sf_ref, pf_ref, out_ref):
    S = s_ref[...]
    Pp = p_ref[...]
    sf = sf_ref[...]
    pf = pf_ref[...]
    acc = jnp.zeros((K, M), jnp.float32)
    for a in range(TOPK):
        pa = Pp[:, a:a + 1]
        sa = S[:, a:a + 1]
        acc = acc + jnp.where(pf == pa, sa * sf, 0.0)
    r = (lax.broadcasted_iota(jnp.int32, (M, K), 0) // TOPK ==
         lax.broadcasted_iota(jnp.int32, (M, K), 1)).astype(jnp.float32)
    gram = jnp.dot(acc, r, preferred_element_type=jnp.float32)
    ncol = jnp.sum(S * S, axis=1, keepdims=True)
    nrow = jnp.dot(sf * sf, r, preferred_element_type=jnp.float32)
    d = jnp.maximum(ncol + nrow - 2.0 * gram, 0.0)
    t = jnp.sum(d, axis=1, keepdims=True)
    out_ref[...] = 0.5 * jnp.sum(t, axis=0, keepdims=True) / (K * K)


def _k5_body(alpha_ref, z_ref, e_ref, db0_ref, db1_ref, re_ref, th_ref):
    i = pl.program_id(0)
    a = alpha_ref[...]
    mx = jnp.max(a, axis=1, keepdims=True)
    ex = jnp.exp(a - mx)
    th = ex / jnp.sum(ex, axis=1, keepdims=True)

    @pl.when(i == 0)
    def _():
        th_ref[...] = th

    tp = th * (1.0 / z_ref[...])
    ev = jnp.concatenate([e_ref[:, g, :] for g in range(GPT5)], axis=1)
    H = B // 2
    col = i * VT5 + lax.broadcasted_iota(jnp.int32, (H, VT5), 1)
    valid = col < V
    accs = []
    for h, db_ref in ((0, db0_ref), (1, db1_ref)):
        p = jnp.dot(tp[h * H:(h + 1) * H, :].astype(jnp.bfloat16),
                    ev.astype(jnp.bfloat16),
                    preferred_element_type=jnp.float32)
        lg = jnp.log(jnp.where(valid, p, 1.0))
        contrib = lg * jnp.where(valid, db_ref[...], 0.0)
        accs.append(jnp.sum(contrib, axis=1, keepdims=True))
    acc = jnp.concatenate(accs, axis=0)

    @pl.when(i == 0)
    def _():
        re_ref[...] = -acc

    @pl.when(i > 0)
    def _():
        re_ref[...] = re_ref[...] - acc


def _sc_gather(table, idx, nrows, nc):
    nw = nc * 16
    bpw = nrows // nw
    mesh = plsc.VectorSubcoreMesh(core_axis_name="c", subcore_axis_name="s")

    @functools.partial(
        pl.kernel, mesh=mesh,
        out_type=jax.ShapeDtypeStruct((nrows, G), jnp.float32),
        scratch_types=[
            pltpu.VMEM((bpw,), jnp.int32),
            pltpu.VMEM((bpw, G), jnp.float32),
            pltpu.SemaphoreType.DMA,
        ],
    )
    def k(table_hbm, idx_hbm, out_hbm, idx_v, rows_v, sem):
        wid = lax.axis_index("s") * nc + lax.axis_index("c")
        base = wid * bpw
        pltpu.sync_copy(idx_hbm.at[pl.ds(base, bpw)], idx_v)
        pltpu.async_copy(table_hbm.at[idx_v], rows_v, sem).wait()
        pltpu.sync_copy(rows_v, out_hbm.at[pl.ds(base, bpw)])

    return k(table, idx)


def kernel(alpha, doc_bow, topic_embeddings_mat, word_embeddings_mat):
    te = topic_embeddings_mat
    we = word_embeddings_mat

    e, gmax3, z = pl.pallas_call(
        _k1_body,
        grid=(NT,),
        in_specs=[
            pl.BlockSpec((K, E), lambda i: (0, 0)),
            pl.BlockSpec((E, VT), lambda i: (0, i)),
        ],
        out_specs=[
            pl.BlockSpec((K, GPT, G), lambda i: (0, i, 0)),
            pl.BlockSpec((1, K, GPT), lambda i: (i, 0, 0)),
            pl.BlockSpec((K, 1), lambda i: (0, 0)),
        ],
        out_shape=[
            jax.ShapeDtypeStruct((K, NG, G), jnp.float32),
            jax.ShapeDtypeStruct((NT, K, GPT), jnp.float32),
            jax.ShapeDtypeStruct((K, 1), jnp.float32),
        ],
    )(te, we)

    gmax = gmax3.transpose(1, 0, 2).reshape(K, NG)
    gid = pl.pallas_call(
        _k2_body,
        out_shape=jax.ShapeDtypeStruct((K, TOPK), jnp.int32),
    )(gmax)

    info = plsc.get_sparse_core_info()
    nc = info.num_cores
    align = 8 * nc * 16
    nrows = ((M + align - 1) // align) * align
    rowid = jnp.arange(K, dtype=jnp.int32)[:, None] * NG + gid
    idx = jnp.concatenate(
        [rowid.reshape(-1), jnp.zeros((nrows - M,), jnp.int32)])
    table = e.reshape(K * NG, G)
    cand = _sc_gather(table, idx, nrows, nc)[:M].reshape(K, TOPK * G)
    cols = (gid[:, :, None] * G +
            jnp.arange(G, dtype=jnp.int32)[None, None, :]).reshape(K, TOPK * G)

    S, poss = pl.pallas_call(
        _k4a_body,
        out_shape=[
            jax.ShapeDtypeStruct((K, TOPK), jnp.float32),
            jax.ShapeDtypeStruct((K, TOPK), jnp.int32),
        ],
    )(cand, cols)

    stdr = pl.pallas_call(
        _k4b_body,
        out_shape=jax.ShapeDtypeStruct((1, 1), jnp.float32),
    )(S, poss, S.reshape(1, M), poss.reshape(1, M))

    zrow = z.reshape(1, K)
    re, theta = pl.pallas_call(
        _k5_body,
        grid=(NT5,),
        in_specs=[
            pl.BlockSpec((B, K), lambda i: (0, 0)),
            pl.BlockSpec((1, K), lambda i: (0, 0)),
            pl.BlockSpec((K, GPT5, G), lambda i: (0, i, 0)),
            pl.BlockSpec((B // 2, VT5), lambda i: (0, i)),
            pl.BlockSpec((B // 2, VT5), lambda i: (1, i)),
        ],
        out_specs=[
            pl.BlockSpec((B, 1), lambda i: (0, 0)),
            pl.BlockSpec((B, K), lambda i: (0, 0)),
        ],
        out_shape=[
            jax.ShapeDtypeStruct((B, 1), jnp.float32),
            jax.ShapeDtypeStruct((B, K), jnp.float32),
        ],
    )(alpha, zrow, e, doc_bow, doc_bow)

    return (re.reshape(B), stdr.reshape(()), theta)


# speedup vs baseline: 1.0334x; 1.0204x over previous
import functools

import jax
import jax.numpy as jnp
from jax import lax
from jax.experimental import pallas as pl
from jax.experimental.pallas import tpu as pltpu
from jax.experimental.pallas import tpu_sc as plsc

B = 1024
K = 128
E = 256
V = 100000
TOPK = 25
VT = 2048
NT = 49
VP = NT * VT
G = 128
NG = VP // G
GPT = VT // G
M = K * TOPK
H = B // 2
NGP = NG + GPT


def _km_body(te_ref, we_ref, alpha_ref, db0_ref, db1_ref,
             table_ref, z_ref, gid_ref, re_ref, th_ref,
             e_scr, gmax_scr):
    p = pl.program_id(0)
    i = pl.program_id(1)

    @pl.when(p == 0)
    def _phase0():
        wt = jnp.dot(te_ref[...], we_ref[...],
                     preferred_element_type=jnp.float32)
        col = i * VT + lax.broadcasted_iota(jnp.int32, (K, VT), 1)
        e = jnp.where(col < V, jnp.exp(wt), 0.0)
        e_scr[i] = e.astype(jnp.bfloat16)
        gi = lax.broadcasted_iota(jnp.int32, (K, GPT), 1)
        gm = jnp.zeros((K, GPT), jnp.float32)
        for g in range(GPT):
            seg = e[:, g * G:(g + 1) * G]
            table_ref[:, g, :] = seg
            gm = jnp.where(gi == g, jnp.max(seg, axis=1, keepdims=True), gm)
        gmax_scr[i] = gm
        zp = jnp.sum(e, axis=1, keepdims=True)

        @pl.when(i == 0)
        def _():
            z_ref[...] = zp

        @pl.when(i > 0)
        def _():
            z_ref[...] = z_ref[...] + zp

    @pl.when(p == 1)
    def _phase1():

        @pl.when(i == 0)
        def _select_groups():
            gm = jnp.concatenate(
                [gmax_scr[t] for t in range(NT)], axis=1)
            gi = lax.broadcasted_iota(jnp.int32, (K, NG), 1)
            oi = lax.broadcasted_iota(jnp.int32, (K, TOPK), 1)
            gid = jnp.zeros((K, TOPK), jnp.int32)
            for j in range(TOPK):
                m = jnp.max(gm, axis=1, keepdims=True)
                pos = jnp.min(jnp.where(gm == m, gi, 2**30), axis=1,
                              keepdims=True)
                gid = jnp.where(oi == j, pos, gid)
                gm = jnp.where(gi == pos, -1.0, gm)
            gid_ref[...] = gid

        a = alpha_ref[...]
        mx = jnp.max(a, axis=1, keepdims=True)
        ex = jnp.exp(a - mx)
        th = ex / jnp.sum(ex, axis=1, keepdims=True)

        @pl.when(i == 0)
        def _():
            th_ref[...] = th

        zinv = 1.0 / z_ref[...]
        ev = e_scr[i]
        evs = (ev.astype(jnp.float32) * zinv).astype(jnp.bfloat16)
        th16 = th.astype(jnp.bfloat16)
        col = i * VT + lax.broadcasted_iota(jnp.int32, (H, VT), 1)
        valid = col < V
        accs = []
        for h, db_ref in ((0, db0_ref), (1, db1_ref)):
            pr = jnp.dot(th16[h * H:(h + 1) * H, :], evs,
                         preferred_element_type=jnp.float32)
            lg = jnp.log(jnp.where(valid, pr, 1.0))
            contrib = lg * jnp.where(valid, db_ref[...], 0.0)
            accs.append(jnp.sum(contrib, axis=1, keepdims=True))
        acc = jnp.concatenate(accs, axis=0)

        @pl.when(i == 0)
        def _():
            re_ref[...] = -acc

        @pl.when(i > 0)
        def _():
            re_ref[...] = re_ref[...] - acc


def _k4a_body(cand_ref, cols_ref, s_ref, p_ref):
    c = cand_ref[...]
    colv = cols_ref[...]
    oi = lax.broadcasted_iota(jnp.int32, (K, TOPK), 1)
    vals = jnp.zeros((K, TOPK), jnp.float32)
    poss = jnp.zeros((K, TOPK), jnp.int32)
    for j in range(TOPK):
        m = jnp.max(c, axis=1, keepdims=True)
        pos = jnp.min(jnp.where(c == m, colv, 2**30), axis=1, keepdims=True)
        vals = jnp.where(oi == j, m, vals)
        poss = jnp.where(oi == j, pos, poss)
        c = jnp.where(colv == pos, -1.0, c)
    s_ref[...] = vals / jnp.sum(vals, axis=1, keepdims=True)
    p_ref[...] = poss


def _k4b_body(s_ref, p_ref, sf_ref, pf_ref, out_ref):
    S = s_ref[...]
    Pp = p_ref[...]
    sf = sf_ref[...]
    pf = pf_ref[...]
    acc = jnp.zeros((K, M), jnp.float32)
    for a in range(TOPK):
        pa = Pp[:, a:a + 1]
        sa = S[:, a:a + 1]
        acc = acc + jnp.where(pf == pa, sa * sf, 0.0)
    r = (lax.broadcasted_iota(jnp.int32, (M, K), 0) // TOPK ==
         lax.broadcasted_iota(jnp.int32, (M, K), 1)).astype(jnp.float32)
    gram = jnp.dot(acc, r, preferred_element_type=jnp.float32)
    ncol = jnp.sum(S * S, axis=1, keepdims=True)
    nrow = jnp.dot(sf * sf, r, preferred_element_type=jnp.float32)
    d = jnp.maximum(ncol + nrow - 2.0 * gram, 0.0)
    t = jnp.sum(d, axis=1, keepdims=True)
    out_ref[...] = 0.5 * jnp.sum(t, axis=0, keepdims=True) / (K * K)


def _sc_gather(table, idx, nrows, nc):
    nw = nc * 16
    bpw = nrows // nw
    mesh = plsc.VectorSubcoreMesh(core_axis_name="c", subcore_axis_name="s")

    @functools.partial(
        pl.kernel, mesh=mesh,
        out_type=jax.ShapeDtypeStruct((nrows, G), jnp.float32),
        scratch_types=[
            pltpu.VMEM((bpw,), jnp.int32),
            pltpu.VMEM((bpw, G), jnp.float32),
            pltpu.SemaphoreType.DMA,
        ],
    )
    def k(table_hbm, idx_hbm, out_hbm, idx_v, rows_v, sem):
        wid = lax.axis_index("s") * nc + lax.axis_index("c")
        base = wid * bpw
        pltpu.sync_copy(idx_hbm.at[pl.ds(base, bpw)], idx_v)
        pltpu.async_copy(table_hbm.at[idx_v], rows_v, sem).wait()
        pltpu.sync_copy(rows_v, out_hbm.at[pl.ds(base, bpw)])

    return k(table, idx)


def kernel(alpha, doc_bow, topic_embeddings_mat, word_embeddings_mat):
    te = topic_embeddings_mat
    we = word_embeddings_mat

    table, z, gid, re, theta = pl.pallas_call(
        _km_body,
        grid=(2, NT),
        in_specs=[
            pl.BlockSpec((K, E), lambda p, i: (0, 0)),
            pl.BlockSpec((E, VT), lambda p, i: (0, i * (1 - p))),
            pl.BlockSpec((B, K), lambda p, i: (0, 0)),
            pl.BlockSpec((H, VT), lambda p, i: (0, i * p)),
            pl.BlockSpec((H, VT), lambda p, i: (1, i * p)),
        ],
        out_specs=[
            pl.BlockSpec((K, GPT, G), lambda p, i: (0, i * (1 - p) + NT * p, 0)),
            pl.BlockSpec((K, 1), lambda p, i: (0, 0)),
            pl.BlockSpec((K, TOPK), lambda p, i: (0, 0)),
            pl.BlockSpec((B, 1), lambda p, i: (0, 0)),
            pl.BlockSpec((B, K), lambda p, i: (0, 0)),
        ],
        out_shape=[
            jax.ShapeDtypeStruct((K, NGP, G), jnp.float32),
            jax.ShapeDtypeStruct((K, 1), jnp.float32),
            jax.ShapeDtypeStruct((K, TOPK), jnp.int32),
            jax.ShapeDtypeStruct((B, 1), jnp.float32),
            jax.ShapeDtypeStruct((B, K), jnp.float32),
        ],
        scratch_shapes=[
            pltpu.VMEM((NT, K, VT), jnp.bfloat16),
            pltpu.VMEM((NT, K, GPT), jnp.float32),
        ],
    )(te, we, alpha, doc_bow, doc_bow)

    info = plsc.get_sparse_core_info()
    nc = info.num_cores
    align = 8 * nc * 16
    nrows = ((M + align - 1) // align) * align
    rowid = jnp.arange(K, dtype=jnp.int32)[:, None] * NGP + gid
    idx = jnp.concatenate(
        [rowid.reshape(-1), jnp.zeros((nrows - M,), jnp.int32)])
    tbl = table.reshape(K * NGP, G)
    cand = _sc_gather(tbl, idx, nrows, nc)[:M].reshape(K, TOPK * G)
    cols = (gid[:, :, None] * G +
            jnp.arange(G, dtype=jnp.int32)[None, None, :]).reshape(K, TOPK * G)

    S, poss = pl.pallas_call(
        _k4a_body,
        out_shape=[
            jax.ShapeDtypeStruct((K, TOPK), jnp.float32),
            jax.ShapeDtypeStruct((K, TOPK), jnp.int32),
        ],
    )(cand, cols)

    stdr = pl.pallas_call(
        _k4b_body,
        out_shape=jax.ShapeDtypeStruct((1, 1), jnp.float32),
    )(S, poss, S.reshape(1, M), poss.reshape(1, M))

    return (re.reshape(B), stdr.reshape(()), theta)


# speedup vs baseline: 1.0335x; 1.0002x over previous
import functools

import jax
import jax.numpy as jnp
from jax import lax
from jax.experimental import pallas as pl
from jax.experimental.pallas import tpu as pltpu
from jax.experimental.pallas import tpu_sc as plsc

B = 1024
K = 128
E = 256
V = 100000
TOPK = 25
VT = 2048
NT = 49
VP = NT * VT
G = 128
NG = VP // G
GPT = VT // G
M = K * TOPK
H = B // 2
NGP = NG + GPT
SDMA = 4
RS = B // SDMA
VLAST = 1664
VMAIN = (NT - 1) * VT + VLAST
TAILB = VMAIN // G


def _db_descs(db_hbm, dbuf, sems, it, width):
    slot = lax.rem(it, 2)
    col0 = it * VT
    return [
        pltpu.make_async_copy(
            db_hbm.at[pl.ds(s * RS, RS), pl.ds(col0, width)],
            dbuf.at[slot, pl.ds(s * RS, RS), pl.ds(0, width)],
            sems.at[slot, s])
        for s in range(SDMA)
    ]


def _db_io(db_hbm, dbuf, sems, it, op):
    @pl.when(it < NT - 1)
    def _():
        for c in _db_descs(db_hbm, dbuf, sems, it, VT):
            getattr(c, op)()

    @pl.when(it == NT - 1)
    def _():
        for c in _db_descs(db_hbm, dbuf, sems, it, VLAST):
            getattr(c, op)()


def _db_start(db_hbm, dbuf, sems, it):
    _db_io(db_hbm, dbuf, sems, it, "start")


def _db_wait(db_hbm, dbuf, sems, it):
    _db_io(db_hbm, dbuf, sems, it, "wait")


def _km_body(te_ref, we_ref, alpha_ref, db_hbm, dbt_ref,
             table_ref, z_ref, gid_ref, re_ref, th_ref,
             e_scr, gmax_scr, dbuf, sems):
    p = pl.program_id(0)
    i = pl.program_id(1)

    @pl.when(p == 0)
    def _phase0():
        wt = jnp.dot(te_ref[...], we_ref[...],
                     preferred_element_type=jnp.float32)
        col = i * VT + lax.broadcasted_iota(jnp.int32, (K, VT), 1)
        e = jnp.where(col < V, jnp.exp(wt), 0.0)
        e_scr[i] = e.astype(jnp.bfloat16)
        gi = lax.broadcasted_iota(jnp.int32, (K, GPT), 1)
        gm = jnp.zeros((K, GPT), jnp.float32)
        for g in range(GPT):
            seg = e[:, g * G:(g + 1) * G]
            table_ref[:, g, :] = seg
            gm = jnp.where(gi == g, jnp.max(seg, axis=1, keepdims=True), gm)
        gmax_scr[i] = gm
        zp = jnp.sum(e, axis=1, keepdims=True)

        @pl.when(i == 0)
        def _():
            z_ref[...] = zp

        @pl.when(i > 0)
        def _():
            z_ref[...] = z_ref[...] + zp

        @pl.when(i == NT - 1)
        def _prefetch_db():
            _db_start(db_hbm, dbuf, sems, 0)

    @pl.when(p == 1)
    def _phase1():

        @pl.when(i == 0)
        def _select_groups():
            gm = jnp.concatenate(
                [gmax_scr[t] for t in range(NT)], axis=1)
            gi = lax.broadcasted_iota(jnp.int32, (K, NG), 1)
            oi = lax.broadcasted_iota(jnp.int32, (K, TOPK), 1)
            gid = jnp.zeros((K, TOPK), jnp.int32)
            for j in range(TOPK):
                m = jnp.max(gm, axis=1, keepdims=True)
                pos = jnp.min(jnp.where(gm == m, gi, 2**30), axis=1,
                              keepdims=True)
                gid = jnp.where(oi == j, pos, gid)
                gm = jnp.where(gi == pos, -1.0, gm)
            gid_ref[...] = gid

        a = alpha_ref[...]
        mx = jnp.max(a, axis=1, keepdims=True)
        ex = jnp.exp(a - mx)
        th = ex / jnp.sum(ex, axis=1, keepdims=True)

        @pl.when(i == 0)
        def _():
            th_ref[...] = th

        @pl.when(i + 1 < NT)
        def _start_next():
            _db_start(db_hbm, dbuf, sems, i + 1)

        _db_wait(db_hbm, dbuf, sems, i)

        zinv = 1.0 / z_ref[...]
        ev = e_scr[i]
        evs = (ev.astype(jnp.float32) * zinv).astype(jnp.bfloat16)
        th16 = th.astype(jnp.bfloat16)
        col = i * VT + lax.broadcasted_iota(jnp.int32, (H, VT), 1)
        valid = col < V
        slot = lax.rem(i, 2)
        vmain = col < VMAIN
        tcol = VMAIN + lax.broadcasted_iota(jnp.int32, (H, G), 1)
        tvalid = tcol < V
        is_last = i == NT - 1
        accs = []
        for h in (0, 1):
            db = dbuf[slot, h * H:(h + 1) * H, :]
            pr = jnp.dot(th16[h * H:(h + 1) * H, :], evs,
                         preferred_element_type=jnp.float32)
            lg = jnp.log(jnp.where(valid, pr, 1.0))
            contrib = lg * jnp.where(vmain, db, 0.0)
            a = jnp.sum(contrib, axis=1, keepdims=True)
            dbt = dbt_ref[h * H:(h + 1) * H, :]
            tc = lg[:, VLAST:VLAST + G] * jnp.where(tvalid, dbt, 0.0)
            a = a + jnp.where(is_last,
                              jnp.sum(tc, axis=1, keepdims=True), 0.0)
            accs.append(a)
        acc = jnp.concatenate(accs, axis=0)

        @pl.when(i == 0)
        def _():
            re_ref[...] = -acc

        @pl.when(i > 0)
        def _():
            re_ref[...] = re_ref[...] - acc


def _k4a_body(cand_ref, cols_ref, s_ref, p_ref):
    c = cand_ref[...]
    colv = cols_ref[...]
    oi = lax.broadcasted_iota(jnp.int32, (K, TOPK), 1)
    vals = jnp.zeros((K, TOPK), jnp.float32)
    poss = jnp.zeros((K, TOPK), jnp.int32)
    for j in range(TOPK):
        m = jnp.max(c, axis=1, keepdims=True)
        pos = jnp.min(jnp.where(c == m, colv, 2**30), axis=1, keepdims=True)
        vals = jnp.where(oi == j, m, vals)
        poss = jnp.where(oi == j, pos, poss)
        c = jnp.where(colv == pos, -1.0, c)
    s_ref[...] = vals / jnp.sum(vals, axis=1, keepdims=True)
    p_ref[...] = poss


def _k4b_body(s_ref, p_ref, sf_ref, pf_ref, out_ref):
    S = s_ref[...]
    Pp = p_ref[...]
    sf = sf_ref[...]
    pf = pf_ref[...]
    acc = jnp.zeros((K, M), jnp.float32)
    for a in range(TOPK):
        pa = Pp[:, a:a + 1]
        sa = S[:, a:a + 1]
        acc = acc + jnp.where(pf == pa, sa * sf, 0.0)
    r = (lax.broadcasted_iota(jnp.int32, (M, K), 0) // TOPK ==
         lax.broadcasted_iota(jnp.int32, (M, K), 1)).astype(jnp.float32)
    gram = jnp.dot(acc, r, preferred_element_type=jnp.float32)
    ncol = jnp.sum(S * S, axis=1, keepdims=True)
    nrow = jnp.dot(sf * sf, r, preferred_element_type=jnp.float32)
    d = jnp.maximum(ncol + nrow - 2.0 * gram, 0.0)
    t = jnp.sum(d, axis=1, keepdims=True)
    out_ref[...] = 0.5 * jnp.sum(t, axis=0, keepdims=True) / (K * K)


def _sc_gather(table, idx, nrows, nc):
    nw = nc * 16
    bpw = nrows // nw
    mesh = plsc.VectorSubcoreMesh(core_axis_name="c", subcore_axis_name="s")

    @functools.partial(
        pl.kernel, mesh=mesh,
        out_type=jax.ShapeDtypeStruct((nrows, G), jnp.float32),
        scratch_types=[
            pltpu.VMEM((bpw,), jnp.int32),
            pltpu.VMEM((bpw, G), jnp.float32),
            pltpu.SemaphoreType.DMA,
        ],
    )
    def k(table_hbm, idx_hbm, out_hbm, idx_v, rows_v, sem):
        wid = lax.axis_index("s") * nc + lax.axis_index("c")
        base = wid * bpw
        pltpu.sync_copy(idx_hbm.at[pl.ds(base, bpw)], idx_v)
        pltpu.async_copy(table_hbm.at[idx_v], rows_v, sem).wait()
        pltpu.sync_copy(rows_v, out_hbm.at[pl.ds(base, bpw)])

    return k(table, idx)


def kernel(alpha, doc_bow, topic_embeddings_mat, word_embeddings_mat):
    te = topic_embeddings_mat
    we = word_embeddings_mat

    table, z, gid, re, theta = pl.pallas_call(
        _km_body,
        grid=(2, NT),
        in_specs=[
            pl.BlockSpec((K, E), lambda p, i: (0, 0)),
            pl.BlockSpec((E, VT), lambda p, i: (0, i * (1 - p))),
            pl.BlockSpec((B, K), lambda p, i: (0, 0)),
            pl.BlockSpec(memory_space=pltpu.MemorySpace.HBM),
            pl.BlockSpec((B, G), lambda p, i: (0, TAILB)),
        ],
        out_specs=[
            pl.BlockSpec((K, GPT, G), lambda p, i: (0, i * (1 - p) + NT * p, 0)),
            pl.BlockSpec((K, 1), lambda p, i: (0, 0)),
            pl.BlockSpec((K, TOPK), lambda p, i: (0, 0)),
            pl.BlockSpec((B, 1), lambda p, i: (0, 0)),
            pl.BlockSpec((B, K), lambda p, i: (0, 0)),
        ],
        out_shape=[
            jax.ShapeDtypeStruct((K, NGP, G), jnp.float32),
            jax.ShapeDtypeStruct((K, 1), jnp.float32),
            jax.ShapeDtypeStruct((K, TOPK), jnp.int32),
            jax.ShapeDtypeStruct((B, 1), jnp.float32),
            jax.ShapeDtypeStruct((B, K), jnp.float32),
        ],
        scratch_shapes=[
            pltpu.VMEM((NT, K, VT), jnp.bfloat16),
            pltpu.VMEM((NT, K, GPT), jnp.float32),
            pltpu.VMEM((2, B, VT), jnp.float32),
            pltpu.SemaphoreType.DMA((2, SDMA)),
        ],
    )(te, we, alpha, doc_bow, doc_bow)

    info = plsc.get_sparse_core_info()
    nc = info.num_cores
    align = 8 * nc * 16
    nrows = ((M + align - 1) // align) * align
    rowid = jnp.arange(K, dtype=jnp.int32)[:, None] * NGP + gid
    idx = jnp.concatenate(
        [rowid.reshape(-1), jnp.zeros((nrows - M,), jnp.int32)])
    tbl = table.reshape(K * NGP, G)
    cand = _sc_gather(tbl, idx, nrows, nc)[:M].reshape(K, TOPK * G)
    cols = (gid[:, :, None] * G +
            jnp.arange(G, dtype=jnp.int32)[None, None, :]).reshape(K, TOPK * G)

    S, poss = pl.pallas_call(
        _k4a_body,
        out_shape=[
            jax.ShapeDtypeStruct((K, TOPK), jnp.float32),
            jax.ShapeDtypeStruct((K, TOPK), jnp.int32),
        ],
    )(cand, cols)

    stdr = pl.pallas_call(
        _k4b_body,
        out_shape=jax.ShapeDtypeStruct((1, 1), jnp.float32),
    )(S, poss, S.reshape(1, M), poss.reshape(1, M))

    return (re.reshape(B), stdr.reshape(()), theta)


# speedup vs baseline: 1.0398x; 1.0060x over previous
import functools

import jax
import jax.numpy as jnp
from jax import lax
from jax.experimental import pallas as pl
from jax.experimental.pallas import tpu as pltpu
from jax.experimental.pallas import tpu_sc as plsc

B = 1024
K = 128
E = 256
V = 100000
TOPK = 25
VT = 2048
NT = 49
VP = NT * VT
G = 128
NG = VP // G
GPT = VT // G
M = K * TOPK
H = B // 2
NGP = NG + GPT
SDMA = 4
RS = B // SDMA
VLAST = 1664
VMAIN = (NT - 1) * VT + VLAST
TAILB = VMAIN // G


def _db_descs(db_hbm, dbuf, sems, it, width):
    slot = lax.rem(it, 2)
    col0 = it * VT
    return [
        pltpu.make_async_copy(
            db_hbm.at[pl.ds(s * RS, RS), pl.ds(col0, width)],
            dbuf.at[slot, pl.ds(s * RS, RS), pl.ds(0, width)],
            sems.at[slot, s])
        for s in range(SDMA)
    ]


def _db_io(db_hbm, dbuf, sems, it, op):
    @pl.when(it < NT - 1)
    def _():
        for c in _db_descs(db_hbm, dbuf, sems, it, VT):
            getattr(c, op)()

    @pl.when(it == NT - 1)
    def _():
        for c in _db_descs(db_hbm, dbuf, sems, it, VLAST):
            getattr(c, op)()


def _db_start(db_hbm, dbuf, sems, it):
    _db_io(db_hbm, dbuf, sems, it, "start")


def _db_wait(db_hbm, dbuf, sems, it):
    _db_io(db_hbm, dbuf, sems, it, "wait")


def _km_body(te_ref, we_ref, alpha_ref, db_hbm, dbt_ref,
             table_ref, z_ref, gid_ref, re_ref, th_ref,
             e_scr, gmax_scr, dbuf, sems):
    p = pl.program_id(0)
    i = pl.program_id(1)

    @pl.when(p == 0)
    def _phase0():
        wt = jnp.dot(te_ref[...], we_ref[...],
                     preferred_element_type=jnp.float32)
        col = i * VT + lax.broadcasted_iota(jnp.int32, (K, VT), 1)
        e = jnp.where(col < V, jnp.exp(wt), 0.0)
        e_scr[i] = e.astype(jnp.bfloat16)
        gi = lax.broadcasted_iota(jnp.int32, (K, GPT), 1)
        gm = jnp.zeros((K, GPT), jnp.float32)
        for g in range(GPT):
            seg = e[:, g * G:(g + 1) * G]
            table_ref[:, g, :] = seg
            gm = jnp.where(gi == g, jnp.max(seg, axis=1, keepdims=True), gm)
        gmax_scr[i] = gm
        zp = jnp.sum(e, axis=1, keepdims=True)

        @pl.when(i == 0)
        def _():
            z_ref[...] = zp

        @pl.when(i > 0)
        def _():
            z_ref[...] = z_ref[...] + zp

        @pl.when(i == NT - 1)
        def _prefetch_db():
            _db_start(db_hbm, dbuf, sems, 0)

    @pl.when(p == 1)
    def _phase1():

        @pl.when(i == 0)
        def _select_groups():
            gm = jnp.concatenate(
                [gmax_scr[t] for t in range(NT)], axis=1)
            gi = lax.broadcasted_iota(jnp.int32, (K, NG), 1)
            oi = lax.broadcasted_iota(jnp.int32, (K, TOPK), 1)
            gid = jnp.zeros((K, TOPK), jnp.int32)
            for j in range(TOPK):
                m = jnp.max(gm, axis=1, keepdims=True)
                pos = jnp.min(jnp.where(gm == m, gi, 2**30), axis=1,
                              keepdims=True)
                gid = jnp.where(oi == j, pos, gid)
                gm = jnp.where(gi == pos, -1.0, gm)
            gid_ref[...] = gid

        a = alpha_ref[...]
        mx = jnp.max(a, axis=1, keepdims=True)
        ex = jnp.exp(a - mx)
        th = ex / jnp.sum(ex, axis=1, keepdims=True)

        @pl.when(i == 0)
        def _():
            th_ref[...] = th

        @pl.when(i + 1 < NT)
        def _start_next():
            _db_start(db_hbm, dbuf, sems, i + 1)

        _db_wait(db_hbm, dbuf, sems, i)

        zinv = 1.0 / z_ref[...]
        ev = e_scr[i]
        evs = (ev.astype(jnp.float32) * zinv).astype(jnp.bfloat16)
        th16 = th.astype(jnp.bfloat16)
        col = i * VT + lax.broadcasted_iota(jnp.int32, (H, VT), 1)
        valid = col < V
        slot = lax.rem(i, 2)
        vmain = col < VMAIN
        tcol = VMAIN + lax.broadcasted_iota(jnp.int32, (H, G), 1)
        tvalid = tcol < V
        is_last = i == NT - 1
        accs = []
        for h in (0, 1):
            db = dbuf[slot, h * H:(h + 1) * H, :]
            pr = jnp.dot(th16[h * H:(h + 1) * H, :], evs,
                         preferred_element_type=jnp.float32)
            lg = jnp.log(jnp.where(valid, pr, 1.0))
            contrib = lg * jnp.where(vmain, db, 0.0)
            a = jnp.sum(contrib, axis=1, keepdims=True)
            dbt = dbt_ref[h * H:(h + 1) * H, :]
            tc = lg[:, VLAST:VLAST + G] * jnp.where(tvalid, dbt, 0.0)
            a = a + jnp.where(is_last,
                              jnp.sum(tc, axis=1, keepdims=True), 0.0)
            accs.append(a)
        acc = jnp.concatenate(accs, axis=0)

        @pl.when(i == 0)
        def _():
            re_ref[...] = -acc

        @pl.when(i > 0)
        def _():
            re_ref[...] = re_ref[...] - acc


def _k4_body(cand_ref, cols_ref, out_ref):
    c = cand_ref[...]
    colv = cols_ref[...]
    oi = lax.broadcasted_iota(jnp.int32, (K, TOPK), 1)
    vals = jnp.zeros((K, TOPK), jnp.float32)
    poss = jnp.zeros((K, TOPK), jnp.int32)
    for j in range(TOPK):
        m = jnp.max(c, axis=1, keepdims=True)
        pos = jnp.min(jnp.where(c == m, colv, 2**30), axis=1, keepdims=True)
        vals = jnp.where(oi == j, m, vals)
        poss = jnp.where(oi == j, pos, poss)
        c = jnp.where(colv == pos, -1.0, c)
    S = vals / jnp.sum(vals, axis=1, keepdims=True)
    St = jnp.transpose(S)
    Pt = jnp.transpose(poss)
    sf = jnp.concatenate([St[a:a + 1, :] for a in range(TOPK)], axis=1)
    pf = jnp.concatenate([Pt[a:a + 1, :] for a in range(TOPK)], axis=1)
    acc = jnp.zeros((K, M), jnp.float32)
    for a in range(TOPK):
        pa = poss[:, a:a + 1]
        sa = S[:, a:a + 1]
        acc = acc + jnp.where(pf == pa, sa * sf, 0.0)
    r = (lax.broadcasted_iota(jnp.int32, (M, K), 0) % K ==
         lax.broadcasted_iota(jnp.int32, (M, K), 1)).astype(jnp.float32)
    gram = jnp.dot(acc, r, preferred_element_type=jnp.float32)
    ncol = jnp.sum(S * S, axis=1, keepdims=True)
    nrow = jnp.dot(sf * sf, r, preferred_element_type=jnp.float32)
    d = jnp.maximum(ncol + nrow - 2.0 * gram, 0.0)
    t = jnp.sum(d, axis=1, keepdims=True)
    out_ref[...] = 0.5 * jnp.sum(t, axis=0, keepdims=True) / (K * K)


def _sc_gather(table, idx, nrows, nc):
    nw = nc * 16
    bpw = nrows // nw
    mesh = plsc.VectorSubcoreMesh(core_axis_name="c", subcore_axis_name="s")

    @functools.partial(
        pl.kernel, mesh=mesh,
        out_type=jax.ShapeDtypeStruct((nrows, G), jnp.float32),
        scratch_types=[
            pltpu.VMEM((bpw,), jnp.int32),
            pltpu.VMEM((bpw, G), jnp.float32),
            pltpu.SemaphoreType.DMA,
        ],
    )
    def k(table_hbm, idx_hbm, out_hbm, idx_v, rows_v, sem):
        wid = lax.axis_index("s") * nc + lax.axis_index("c")
        base = wid * bpw
        pltpu.sync_copy(idx_hbm.at[pl.ds(base, bpw)], idx_v)
        pltpu.async_copy(table_hbm.at[idx_v], rows_v, sem).wait()
        pltpu.sync_copy(rows_v, out_hbm.at[pl.ds(base, bpw)])

    return k(table, idx)


def kernel(alpha, doc_bow, topic_embeddings_mat, word_embeddings_mat):
    te = topic_embeddings_mat
    we = word_embeddings_mat

    table, z, gid, re, theta = pl.pallas_call(
        _km_body,
        grid=(2, NT),
        in_specs=[
            pl.BlockSpec((K, E), lambda p, i: (0, 0)),
            pl.BlockSpec((E, VT), lambda p, i: (0, i * (1 - p))),
            pl.BlockSpec((B, K), lambda p, i: (0, 0)),
            pl.BlockSpec(memory_space=pltpu.MemorySpace.HBM),
            pl.BlockSpec((B, G), lambda p, i: (0, TAILB)),
        ],
        out_specs=[
            pl.BlockSpec((K, GPT, G), lambda p, i: (0, i * (1 - p) + NT * p, 0)),
            pl.BlockSpec((K, 1), lambda p, i: (0, 0)),
            pl.BlockSpec((K, TOPK), lambda p, i: (0, 0)),
            pl.BlockSpec((B, 1), lambda p, i: (0, 0)),
            pl.BlockSpec((B, K), lambda p, i: (0, 0)),
        ],
        out_shape=[
            jax.ShapeDtypeStruct((K, NGP, G), jnp.float32),
            jax.ShapeDtypeStruct((K, 1), jnp.float32),
            jax.ShapeDtypeStruct((K, TOPK), jnp.int32),
            jax.ShapeDtypeStruct((B, 1), jnp.float32),
            jax.ShapeDtypeStruct((B, K), jnp.float32),
        ],
        scratch_shapes=[
            pltpu.VMEM((NT, K, VT), jnp.bfloat16),
            pltpu.VMEM((NT, K, GPT), jnp.float32),
            pltpu.VMEM((2, B, VT), jnp.float32),
            pltpu.SemaphoreType.DMA((2, SDMA)),
        ],
    )(te, we, alpha, doc_bow, doc_bow)

    info = plsc.get_sparse_core_info()
    nc = info.num_cores
    align = 8 * nc * 16
    nrows = ((M + align - 1) // align) * align
    rowid = jnp.arange(K, dtype=jnp.int32)[:, None] * NGP + gid
    idx = jnp.concatenate(
        [rowid.reshape(-1), jnp.zeros((nrows - M,), jnp.int32)])
    tbl = table.reshape(K * NGP, G)
    cand = _sc_gather(tbl, idx, nrows, nc)[:M].reshape(K, TOPK * G)
    cols = (gid[:, :, None] * G +
            jnp.arange(G, dtype=jnp.int32)[None, None, :]).reshape(K, TOPK * G)

    stdr = pl.pallas_call(
        _k4_body,
        out_shape=jax.ShapeDtypeStruct((1, 1), jnp.float32),
    )(cand, cols)

    return (re.reshape(B), stdr.reshape(()), theta)
